# Initial kernel scaffold; baseline (speedup 1.0000x reference)
import jax
import jax.numpy as jnp
from jax.experimental import pallas as pl


def kernel(x_cont, x_cat, emb_tables, W0, b0, W1, b1, W2, b2, Wh, bh):
    raise NotImplementedError("write your pallas kernel here")


# speedup vs baseline: 2.2127x; 2.2127x over previous
import functools

import jax
import jax.numpy as jnp
from jax import lax
from jax.experimental import pallas as pl
from jax.experimental.pallas import tpu as pltpu
from jax.experimental.pallas import tpu_sc as plsc

N_FIELDS = 26
VOCAB = 100000
EMB_DIM = 16
N_CONT = 13
BATCH = 16384
EMB_FEATS = N_FIELDS * EMB_DIM

_NW = 32
_B_PER_W = BATCH // _NW
_CHUNK_B = 128
_CHUNK_R = _CHUNK_B * N_FIELDS
_N_CHUNKS = _B_PER_W // _CHUNK_B


@functools.cache
def _make_sc_gather():
    info = plsc.get_sparse_core_info()
    num_cores = info.num_cores

    @functools.partial(
        pl.kernel,
        out_type=jax.ShapeDtypeStruct((BATCH * N_FIELDS, EMB_DIM), jnp.float32),
        mesh=plsc.VectorSubcoreMesh(core_axis_name="c", subcore_axis_name="s"),
        scratch_types=[
            pltpu.VMEM((_CHUNK_R,), jnp.int32),
            pltpu.VMEM((_CHUNK_R, EMB_DIM), jnp.float32),
            pltpu.SemaphoreType.DMA,
        ],
        compiler_params=pltpu.CompilerParams(use_tc_tiling_on_sc=False),
    )
    def _sc_gather(table_hbm, idx_hbm, out_hbm, idx_v, rows_v, sem):
        wid = lax.axis_index("s") * num_cores + lax.axis_index("c")
        base = wid * (_B_PER_W * N_FIELDS)
        for g in range(_N_CHUNKS):
            off = base + g * _CHUNK_R
            pltpu.sync_copy(idx_hbm.at[pl.ds(off, _CHUNK_R)], idx_v)
            pltpu.async_copy(table_hbm.at[idx_v], rows_v, sem).wait()
            pltpu.sync_copy(rows_v, out_hbm.at[pl.ds(off, _CHUNK_R)])

    return _sc_gather


def _mlp_body(xe_ref, xc_ref, w0e_ref, w0c_ref, b0_ref, w1_ref, b1_ref,
              w2_ref, b2_ref, wh_ref, bh_ref, out_ref):
    h = jnp.dot(xe_ref[...], w0e_ref[...], preferred_element_type=jnp.float32)
    h += jnp.dot(xc_ref[...], w0c_ref[...], preferred_element_type=jnp.float32)
    h = jnp.maximum(h + b0_ref[...], 0.0)
    h = jnp.maximum(
        jnp.dot(h, w1_ref[...], preferred_element_type=jnp.float32) + b1_ref[...], 0.0)
    h = jnp.maximum(
        jnp.dot(h, w2_ref[...], preferred_element_type=jnp.float32) + b2_ref[...], 0.0)
    out_ref[...] = jnp.dot(h, wh_ref[...], preferred_element_type=jnp.float32) + bh_ref[...]


_BT = 1024


def _mlp(xe, xc, w0e, w0c, b0, w1, b1, w2, b2, wh, bh):
    n_blocks = BATCH // _BT
    full = lambda shape: pl.BlockSpec(shape, lambda i: (0, 0))
    return pl.pallas_call(
        _mlp_body,
        grid=(n_blocks,),
        in_specs=[
            pl.BlockSpec((_BT, EMB_FEATS), lambda i: (i, 0)),
            pl.BlockSpec((_BT, N_CONT), lambda i: (i, 0)),
            full((EMB_FEATS, 512)),
            full((N_CONT, 512)),
            full((1, 512)),
            full((512, 256)),
            full((1, 256)),
            full((256, 128)),
            full((1, 128)),
            full((128, 1)),
            full((1, 1)),
        ],
        out_specs=pl.BlockSpec((_BT, 1), lambda i: (i, 0)),
        out_shape=jax.ShapeDtypeStruct((BATCH, 1), jnp.float32),
    )(xe, xc, w0e, w0c, b0, w1, b1, w2, b2, wh, bh)


def kernel(x_cont, x_cat, emb_tables, W0, b0, W1, b1, W2, b2, Wh, bh):
    table = emb_tables.reshape(N_FIELDS * VOCAB, EMB_DIM)
    offs = jnp.arange(N_FIELDS, dtype=jnp.int32) * VOCAB
    idx = (x_cat.astype(jnp.int32) + offs[None, :]).reshape(-1)
    rows = _make_sc_gather()(table, idx)
    xe = rows.reshape(BATCH, EMB_FEATS)
    return _mlp(xe, x_cont, W0[N_CONT:], W0[:N_CONT],
                b0.reshape(1, -1), W1, b1.reshape(1, -1),
                W2, b2.reshape(1, -1), Wh, bh.reshape(1, 1))
